# baseline (speedup 1.0000x reference)
import functools

import jax
import jax.numpy as jnp
from jax import lax
from jax.experimental import pallas as pl
from jax.experimental.pallas import tpu as pltpu
from jax.experimental.pallas import tpu_sc as plsc

N = 10000
E = 320000
D_IN = 128
D_OUT = 64

_NC = 2
_NS = 16
_NW = _NC * _NS
_EPW = E // _NW
_CHUNK = 80
_NCHUNK = _EPW // _CHUNK
_NPAD = 10240
_RPT = _NPAD // _NS


def _proj_body(feat_ref, w_ref, out_ref):
    y = jnp.dot(feat_ref[...], w_ref[...], preferred_element_type=jnp.float32)
    rows = lax.broadcasted_iota(jnp.int32, (feat_ref.shape[0], 1), 0)
    y = jnp.where((rows % 4) != 0, y, 0.0)
    blk = y.shape[0]
    y = y.reshape(blk, 2, 2, 16).transpose(0, 1, 3, 2).reshape(blk, D_OUT)
    out_ref[...] = y.astype(jnp.bfloat16)


def _project(feat, weight):
    blk = 2000
    return pl.pallas_call(
        _proj_body,
        grid=(N // blk,),
        in_specs=[
            pl.BlockSpec((blk, D_IN), lambda i: (i, 0)),
            pl.BlockSpec((D_IN, D_OUT), lambda i: (0, 0)),
        ],
        out_specs=pl.BlockSpec((blk, D_OUT), lambda i: (i, 0)),
        out_shape=jax.ShapeDtypeStruct((N, D_OUT), jnp.bfloat16),
    )(feat, weight)


def _sc_body(infeat_hbm, ei_hbm, w_hbm, out_hbm,
             src_v, dst_v, w_v, msg_v, msg2_v, msg3_v, msg4_v, fbuf_v, zb_v,
             acc_sh, sem, sem2, sem3, sem4):
    cid = lax.axis_index("c")
    sid = lax.axis_index("s")
    wid = cid * _NS + sid

    def _zrow(r, _):
        for j in range(4):
            zb_v[r, pl.ds(j * 16, 16)] = jnp.zeros((16,), jnp.float32)
        return 0
    lax.fori_loop(0, 128, _zrow, 0)
    base = sid * _RPT
    for t in range(_RPT // 128):
        pltpu.sync_copy(zb_v, acc_sh.at[pl.ds(base + t * 128, 128)])

    pltpu.sync_copy(ei_hbm.at[0, wid], src_v)
    pltpu.sync_copy(ei_hbm.at[1, wid], dst_v)
    pltpu.sync_copy(w_hbm.at[wid], w_v)

    plsc.subcore_barrier()

    def _scale(buf, k):
        for g in range(_CHUNK // 16):
            wv = w_v[k, pl.ds(g * 16, 16)]
            for l in range(16):
                w = wv[l]
                e = g * 16 + l
                for half in range(2):
                    ab = buf[e, pl.ds(half * 32, 32)]
                    lo, hi = plsc.unpack(
                        ab, format=plsc.PackFormat.INTERLEAVED,
                        preferred_element_type=jnp.float32)
                    fbuf_v[e, pl.ds(half * 32, 16)] = lo * w
                    fbuf_v[e, pl.ds(half * 32 + 16, 16)] = hi * w

    def _gather(k, buf, sem):
        pltpu.async_copy(infeat_hbm.at[src_v.at[k]], buf, sem)

    def _gwait(k, buf, sem):
        pltpu.make_async_copy(infeat_hbm.at[src_v.at[k]], buf, sem).wait()

    def _scatter_add(k, buf):
        del buf
        pltpu.sync_copy(fbuf_v, acc_sh.at[dst_v.at[k]], add=True)

    bufs = (msg_v, msg2_v, msg3_v, msg4_v)
    sems = (sem, sem2, sem3, sem4)
    for q in range(4):
        _gather(q, bufs[q], sems[q])

    _MAIN = _NCHUNK // 4

    def _quad(i, _):
        for q in range(4):
            k = 4 * i + q
            _gwait(k, bufs[q], sems[q])
            _scale(bufs[q], k)
            _scatter_add(k, bufs[q])

            @pl.when(k + 4 < _NCHUNK)
            def _():
                _gather(k + 4, bufs[q], sems[q])
        return 0

    lax.fori_loop(0, _MAIN, _quad, 0)
    for k in range(4 * _MAIN, _NCHUNK):
        q = k % 4
        _gwait(k, bufs[q], sems[q])
        _scale(bufs[q], k)
        _scatter_add(k, bufs[q])

    plsc.subcore_barrier()

    pltpu.sync_copy(acc_sh.at[pl.ds(base, _RPT)],
                    out_hbm.at[cid, pl.ds(base, _RPT)])


def _scatter(infeat, ei4, w3):
    mesh = plsc.VectorSubcoreMesh(core_axis_name="c", subcore_axis_name="s", num_cores=_NC)
    kfn = pl.kernel(
        _sc_body,
        out_type=jax.ShapeDtypeStruct((_NC, _NPAD, D_OUT), jnp.float32),
        mesh=mesh,
        scratch_types=[
            pltpu.VMEM((_NCHUNK, _CHUNK), jnp.int32),
            pltpu.VMEM((_NCHUNK, _CHUNK), jnp.int32),
            pltpu.VMEM((_NCHUNK, _CHUNK), jnp.float32),
            pltpu.VMEM((_CHUNK, D_OUT), jnp.bfloat16),
            pltpu.VMEM((_CHUNK, D_OUT), jnp.bfloat16),
            pltpu.VMEM((_CHUNK, D_OUT), jnp.bfloat16),
            pltpu.VMEM((_CHUNK, D_OUT), jnp.bfloat16),
            pltpu.VMEM((_CHUNK, D_OUT), jnp.float32),
            pltpu.VMEM((128, D_OUT), jnp.float32),
            pltpu.VMEM_SHARED((_NPAD, D_OUT), jnp.float32),
            pltpu.SemaphoreType.DMA,
            pltpu.SemaphoreType.DMA,
            pltpu.SemaphoreType.DMA,
            pltpu.SemaphoreType.DMA,
        ],
        compiler_params=pltpu.CompilerParams(use_tc_tiling_on_sc=False, needs_layout_passes=False),
    )
    return kfn(infeat, ei4, w3)


def _l2n(x):
    n = jnp.sqrt(jnp.sum(x * x, axis=1, keepdims=True))
    return x / jnp.maximum(n, 1e-12)


def _epi_body(h4_ref, fa_ref, w_ref, b_ref, pa_ref,
              sw_ref, sb_ref, gw_ref, gb_ref,
              pool_ref, anch_ref, gcn_ref):
    G = N // 4
    a = pa_ref[0, 0]
    b = b_ref[...]
    h = h4_ref[0]
    for c in range(1, _NC):
        h = h + h4_ref[c]
    h = h + b
    h = jnp.where(h >= 0, h, a * h)
    pooled = (h[:, 0, :] + h[:, 1, :] + h[:, 2, :] + h[:, 3, :]) * 0.25
    gcn = h[:G, 0, :]
    anch = jnp.dot(fa_ref[:, 0, :], w_ref[...],
                   preferred_element_type=jnp.float32) + b
    anch = jnp.where(anch >= 0, anch, a * anch)
    pool_ref[...] = _l2n(
        jnp.dot(pooled[:G], sw_ref[...], preferred_element_type=jnp.float32)
        + sb_ref[...])
    anch_ref[...] = _l2n(anch)
    gcn_ref[...] = _l2n(
        jnp.dot(gcn, gw_ref[...], preferred_element_type=jnp.float32)
        + gb_ref[...])


def _epilogue(h4, fa3, weight, bias, pa, subg_W, subg_b, gcn_W, gcn_b):
    G = N // 4
    GP = _NPAD // 4
    out = jax.ShapeDtypeStruct((G, D_OUT), jnp.float32)
    return pl.pallas_call(
        _epi_body,
        in_specs=[
            pl.BlockSpec((_NC, GP, 4, D_OUT), lambda: (0, 0, 0, 0)),
            pl.BlockSpec((G, 4, D_IN), lambda: (0, 0, 0)),
            pl.BlockSpec((D_IN, D_OUT), lambda: (0, 0)),
            pl.BlockSpec((1, D_OUT), lambda: (0, 0)),
            pl.BlockSpec((1, 1), lambda: (0, 0)),
            pl.BlockSpec((D_OUT, D_OUT), lambda: (0, 0)),
            pl.BlockSpec((1, D_OUT), lambda: (0, 0)),
            pl.BlockSpec((D_OUT, D_OUT), lambda: (0, 0)),
            pl.BlockSpec((1, D_OUT), lambda: (0, 0)),
        ],
        out_shape=(out, out, out),
    )(h4, fa3, weight, bias, pa, subg_W, subg_b, gcn_W, gcn_b)


def kernel(feat, edge_index, edge_w, weight, bias, prelu_a,
           subg_W, subg_b, gcn_W, gcn_b):
    infeat = _project(feat, weight)
    ei4 = edge_index.reshape(2, _NW, _NCHUNK, _CHUNK)
    w3 = edge_w.reshape(_NW, _NCHUNK, _CHUNK)
    hpart = _scatter(infeat, ei4, w3)
    h4 = hpart.reshape(_NC, _NPAD // 4, 4, D_OUT)
    fa3 = feat.reshape(N // 4, 4, D_IN)
    pool, anch, gcn = _epilogue(
        h4, fa3, weight, jnp.reshape(bias, (1, D_OUT)),
        jnp.reshape(jnp.asarray(prelu_a, jnp.float32), (1, 1)),
        subg_W, jnp.reshape(subg_b, (1, D_OUT)),
        gcn_W, jnp.reshape(gcn_b, (1, D_OUT)))
    return (pool, anch, gcn)


# speedup vs baseline: 1.8010x; 1.8010x over previous
import functools

import jax
import jax.numpy as jnp
from jax import lax
from jax.experimental import pallas as pl
from jax.experimental.pallas import tpu as pltpu
from jax.experimental.pallas import tpu_sc as plsc

N = 10000
E = 320000
D_IN = 128
D_OUT = 64

_NC = 2
_NS = 16
_NW = _NC * _NS
_EPW = E // _NW
_CHUNK = 80
_NCHUNK = _EPW // _CHUNK
_NPAD = 10240
_RPT = _NPAD // _NS


def _proj_body(feat_ref, w_ref, out_ref):
    y = jnp.dot(feat_ref[...], w_ref[...], preferred_element_type=jnp.float32)
    rows = lax.broadcasted_iota(jnp.int32, (feat_ref.shape[0], 1), 0)
    out_ref[...] = jnp.where((rows % 4) != 0, y, 0.0)


def _project(feat, weight):
    blk = 2000
    return pl.pallas_call(
        _proj_body,
        grid=(N // blk,),
        in_specs=[
            pl.BlockSpec((blk, D_IN), lambda i: (i, 0)),
            pl.BlockSpec((D_IN, D_OUT), lambda i: (0, 0)),
        ],
        out_specs=pl.BlockSpec((blk, D_OUT), lambda i: (i, 0)),
        out_shape=jax.ShapeDtypeStruct((N, D_OUT), jnp.float32),
    )(feat, weight)


def _sc_body(infeat_hbm, ei_hbm, w_hbm, out_hbm,
             src_v, dst_v, w_v, msg_v, msg2_v, msg3_v, msg4_v, ob1_v, ob2_v,
             zb_v, acc_sh, sem, sem2, sem3, sem4, ssem1, ssem2):
    cid = lax.axis_index("c")
    sid = lax.axis_index("s")
    wid = cid * _NS + sid

    def _zrow(r, _):
        for j in range(4):
            zb_v[r, pl.ds(j * 16, 16)] = jnp.zeros((16,), jnp.float32)
        return 0
    lax.fori_loop(0, 128, _zrow, 0)
    base = sid * _RPT
    for t in range(_RPT // 128):
        pltpu.sync_copy(zb_v, acc_sh.at[pl.ds(base + t * 128, 128)])

    pltpu.sync_copy(ei_hbm.at[0, wid], src_v)
    pltpu.sync_copy(ei_hbm.at[1, wid], dst_v)
    pltpu.sync_copy(w_hbm.at[wid], w_v)

    plsc.subcore_barrier()

    def _scale(buf, ob, k):
        for g in range(_CHUNK // 16):
            wv = w_v[k, pl.ds(g * 16, 16)]
            for l in range(16):
                w = wv[l]
                e = g * 16 + l
                for j in range(4):
                    sl = pl.ds(j * 16, 16)
                    ob[e, sl] = buf[e, sl] * w

    def _gather(k, buf, sem):
        pltpu.async_copy(infeat_hbm.at[src_v.at[k]], buf, sem)

    def _gwait(k, buf, sem):
        pltpu.make_async_copy(infeat_hbm.at[src_v.at[k]], buf, sem).wait()

    def _sstart(k, ob, ssem):
        pltpu.async_copy(ob, acc_sh.at[dst_v.at[k]], ssem, add=True)

    def _swait(k, ob, ssem):
        pltpu.make_async_copy(ob, acc_sh.at[dst_v.at[k]], ssem).wait()

    bufs = (msg_v, msg2_v, msg3_v, msg4_v)
    sems = (sem, sem2, sem3, sem4)
    obufs = (ob1_v, ob2_v)
    ssems = (ssem1, ssem2)
    for q in range(4):
        _gather(q, bufs[q], sems[q])

    _MAIN = _NCHUNK // 4

    def _quad(i, _):
        for q in range(4):
            k = 4 * i + q
            p = q % 2
            _gwait(k, bufs[q], sems[q])

            @pl.when(k >= 2)
            def _():
                _swait(k - 2, obufs[p], ssems[p])

            _scale(bufs[q], obufs[p], k)
            _sstart(k, obufs[p], ssems[p])

            @pl.when(k + 4 < _NCHUNK)
            def _():
                _gather(k + 4, bufs[q], sems[q])
        return 0

    lax.fori_loop(0, _MAIN, _quad, 0)
    for k in range(4 * _MAIN, _NCHUNK):
        q = k % 4
        p = k % 2
        _gwait(k, bufs[q], sems[q])
        _swait(k - 2, obufs[p], ssems[p])
        _scale(bufs[q], obufs[p], k)
        _sstart(k, obufs[p], ssems[p])
    for k in (_NCHUNK - 2, _NCHUNK - 1):
        _swait(k, obufs[k % 2], ssems[k % 2])

    plsc.subcore_barrier()

    pltpu.sync_copy(acc_sh.at[pl.ds(base, _RPT)],
                    out_hbm.at[cid, pl.ds(base, _RPT)])


def _scatter(infeat, ei4, w3):
    mesh = plsc.VectorSubcoreMesh(core_axis_name="c", subcore_axis_name="s", num_cores=_NC)
    kfn = pl.kernel(
        _sc_body,
        out_type=jax.ShapeDtypeStruct((_NC, _NPAD, D_OUT), jnp.float32),
        mesh=mesh,
        scratch_types=[
            pltpu.VMEM((_NCHUNK, _CHUNK), jnp.int32),
            pltpu.VMEM((_NCHUNK, _CHUNK), jnp.int32),
            pltpu.VMEM((_NCHUNK, _CHUNK), jnp.float32),
            pltpu.VMEM((_CHUNK, D_OUT), jnp.float32),
            pltpu.VMEM((_CHUNK, D_OUT), jnp.float32),
            pltpu.VMEM((_CHUNK, D_OUT), jnp.float32),
            pltpu.VMEM((_CHUNK, D_OUT), jnp.float32),
            pltpu.VMEM((_CHUNK, D_OUT), jnp.float32),
            pltpu.VMEM((_CHUNK, D_OUT), jnp.float32),
            pltpu.VMEM((128, D_OUT), jnp.float32),
            pltpu.VMEM_SHARED((_NPAD, D_OUT), jnp.float32),
            pltpu.SemaphoreType.DMA,
            pltpu.SemaphoreType.DMA,
            pltpu.SemaphoreType.DMA,
            pltpu.SemaphoreType.DMA,
            pltpu.SemaphoreType.DMA,
            pltpu.SemaphoreType.DMA,
        ],
        compiler_params=pltpu.CompilerParams(use_tc_tiling_on_sc=False),
    )
    return kfn(infeat, ei4, w3)


def _l2n(x):
    n = jnp.sqrt(jnp.sum(x * x, axis=1, keepdims=True))
    return x / jnp.maximum(n, 1e-12)


def _epi_body(h4_ref, fa_ref, w_ref, b_ref, pa_ref,
              sw_ref, sb_ref, gw_ref, gb_ref,
              pool_ref, anch_ref, gcn_ref):
    G = N // 4
    a = pa_ref[0, 0]
    b = b_ref[...]
    h = h4_ref[0]
    for c in range(1, _NC):
        h = h + h4_ref[c]
    h = h + b
    h = jnp.where(h >= 0, h, a * h)
    pooled = (h[:, 0, :] + h[:, 1, :] + h[:, 2, :] + h[:, 3, :]) * 0.25
    gcn = h[:G, 0, :]
    anch = jnp.dot(fa_ref[:, 0, :], w_ref[...],
                   preferred_element_type=jnp.float32) + b
    anch = jnp.where(anch >= 0, anch, a * anch)
    pool_ref[...] = _l2n(
        jnp.dot(pooled[:G], sw_ref[...], preferred_element_type=jnp.float32)
        + sb_ref[...])
    anch_ref[...] = _l2n(anch)
    gcn_ref[...] = _l2n(
        jnp.dot(gcn, gw_ref[...], preferred_element_type=jnp.float32)
        + gb_ref[...])


def _epilogue(h4, fa3, weight, bias, pa, subg_W, subg_b, gcn_W, gcn_b):
    G = N // 4
    GP = _NPAD // 4
    out = jax.ShapeDtypeStruct((G, D_OUT), jnp.float32)
    return pl.pallas_call(
        _epi_body,
        in_specs=[
            pl.BlockSpec((_NC, GP, 4, D_OUT), lambda: (0, 0, 0, 0)),
            pl.BlockSpec((G, 4, D_IN), lambda: (0, 0, 0)),
            pl.BlockSpec((D_IN, D_OUT), lambda: (0, 0)),
            pl.BlockSpec((1, D_OUT), lambda: (0, 0)),
            pl.BlockSpec((1, 1), lambda: (0, 0)),
            pl.BlockSpec((D_OUT, D_OUT), lambda: (0, 0)),
            pl.BlockSpec((1, D_OUT), lambda: (0, 0)),
            pl.BlockSpec((D_OUT, D_OUT), lambda: (0, 0)),
            pl.BlockSpec((1, D_OUT), lambda: (0, 0)),
        ],
        out_shape=(out, out, out),
    )(h4, fa3, weight, bias, pa, subg_W, subg_b, gcn_W, gcn_b)


def kernel(feat, edge_index, edge_w, weight, bias, prelu_a,
           subg_W, subg_b, gcn_W, gcn_b):
    infeat = _project(feat, weight)
    ei4 = edge_index.reshape(2, _NW, _NCHUNK, _CHUNK)
    w3 = edge_w.reshape(_NW, _NCHUNK, _CHUNK)
    hpart = _scatter(infeat, ei4, w3)
    h4 = hpart.reshape(_NC, _NPAD // 4, 4, D_OUT)
    fa3 = feat.reshape(N // 4, 4, D_IN)
    pool, anch, gcn = _epilogue(
        h4, fa3, weight, jnp.reshape(bias, (1, D_OUT)),
        jnp.reshape(jnp.asarray(prelu_a, jnp.float32), (1, 1)),
        subg_W, jnp.reshape(subg_b, (1, D_OUT)),
        gcn_W, jnp.reshape(gcn_b, (1, D_OUT)))
    return (pool, anch, gcn)
